# Initial kernel scaffold; baseline (speedup 1.0000x reference)
import jax
import jax.numpy as jnp
from jax.experimental import pallas as pl


def kernel(indices, table):
    raise NotImplementedError("write your pallas kernel here")


# speedup vs baseline: 3.9580x; 3.9580x over previous
import functools

import jax
import jax.numpy as jnp
from jax import lax
from jax.experimental import pallas as pl
from jax.experimental.pallas import tpu as pltpu
from jax.experimental.pallas import tpu_sc as plsc

_CHUNK = 512


@functools.cache
def _make_gather(B, V, D, n_workers, nc):
    assert B % (n_workers * _CHUNK) == 0
    b_per_w = B // n_workers
    n_chunks = b_per_w // _CHUNK
    mesh = plsc.VectorSubcoreMesh(core_axis_name="c", subcore_axis_name="s")

    @functools.partial(
        pl.kernel,
        mesh=mesh,
        out_type=jax.ShapeDtypeStruct((B, D), jnp.float32),
        scratch_types=[
            pltpu.VMEM((_CHUNK,), jnp.int32),
            pltpu.VMEM((_CHUNK, D), jnp.float32),
            pltpu.SemaphoreType.DMA,
        ],
        compiler_params=pltpu.CompilerParams(use_tc_tiling_on_sc=False),
    )
    def gather_kernel(idx_hbm, table_hbm, out_hbm, idx_v, rows_v, sem):
        wid = lax.axis_index("s") * nc + lax.axis_index("c")
        base = wid * b_per_w

        def body(i, carry):
            off = base + i * _CHUNK
            pltpu.sync_copy(idx_hbm.at[pl.ds(off, _CHUNK)], idx_v)
            pltpu.async_copy(table_hbm.at[idx_v], rows_v, sem).wait()
            pltpu.sync_copy(rows_v, out_hbm.at[pl.ds(off, _CHUNK)])
            return carry

        lax.fori_loop(0, n_chunks, body, 0)

    return gather_kernel


def kernel(indices, table):
    batch, seq = indices.shape
    vocab, dim = table.shape
    info = plsc.get_sparse_core_info()
    n_workers = info.num_cores * info.num_subcores
    flat = indices.reshape(-1)
    out = _make_gather(flat.shape[0], vocab, dim, n_workers, info.num_cores)(
        flat, table
    )
    return out.reshape(batch, seq, dim)


# speedup vs baseline: 4.2363x; 1.0703x over previous
import functools

import jax
import jax.numpy as jnp
from jax import lax
from jax.experimental import pallas as pl
from jax.experimental.pallas import tpu as pltpu
from jax.experimental.pallas import tpu_sc as plsc

_CHUNK = 400
_NBUF = 4


@functools.cache
def _make_gather(B, V, D, n_workers, nc):
    assert B % (n_workers * _CHUNK * _NBUF) == 0
    b_per_w = B // n_workers
    n_chunks = b_per_w // _CHUNK
    n_groups = n_chunks // _NBUF
    mesh = plsc.VectorSubcoreMesh(core_axis_name="c", subcore_axis_name="s")

    scratch = (
        [pltpu.VMEM((_CHUNK,), jnp.int32) for _ in range(_NBUF)]
        + [pltpu.VMEM((_CHUNK, D), jnp.float32) for _ in range(_NBUF)]
        + [pltpu.SemaphoreType.DMA] * (3 * _NBUF)
    )

    @functools.partial(
        pl.kernel,
        mesh=mesh,
        out_type=jax.ShapeDtypeStruct((B, D), jnp.float32),
        scratch_types=scratch,
        compiler_params=pltpu.CompilerParams(use_tc_tiling_on_sc=False),
    )
    def gather_kernel(idx_hbm, table_hbm, out_hbm, *bufs):
        idx_v = bufs[0:_NBUF]
        rows_v = bufs[_NBUF : 2 * _NBUF]
        s_i = bufs[2 * _NBUF : 3 * _NBUF]
        s_g = bufs[3 * _NBUF : 4 * _NBUF]
        s_o = bufs[4 * _NBUF : 5 * _NBUF]

        wid = lax.axis_index("s") * nc + lax.axis_index("c")
        base = wid * b_per_w

        for b in range(_NBUF):
            pltpu.async_copy(
                idx_hbm.at[pl.ds(base + b * _CHUNK, _CHUNK)], idx_v[b], s_i[b]
            )
        for b in range(_NBUF):
            pltpu.make_async_copy(
                idx_hbm.at[pl.ds(base + b * _CHUNK, _CHUNK)], idx_v[b], s_i[b]
            ).wait()
            pltpu.async_copy(table_hbm.at[idx_v[b]], rows_v[b], s_g[b])

        def body(t, carry):
            j0 = t * _NBUF
            for b in range(_NBUF):
                off = base + (j0 + b) * _CHUNK
                pltpu.make_async_copy(
                    table_hbm.at[idx_v[b]], rows_v[b], s_g[b]
                ).wait()
                pltpu.async_copy(rows_v[b], out_hbm.at[pl.ds(off, _CHUNK)], s_o[b])
                pltpu.async_copy(
                    idx_hbm.at[pl.ds(off + _NBUF * _CHUNK, _CHUNK)],
                    idx_v[b],
                    s_i[b],
                )
            for b in range(_NBUF):
                off = base + (j0 + b) * _CHUNK
                pltpu.make_async_copy(
                    idx_hbm.at[pl.ds(off + _NBUF * _CHUNK, _CHUNK)],
                    idx_v[b],
                    s_i[b],
                ).wait()
                pltpu.make_async_copy(
                    rows_v[b], out_hbm.at[pl.ds(off, _CHUNK)], s_o[b]
                ).wait()
                pltpu.async_copy(table_hbm.at[idx_v[b]], rows_v[b], s_g[b])
            return carry

        lax.fori_loop(0, n_groups - 1, body, 0)

        last = base + (n_chunks - _NBUF) * _CHUNK
        for b in range(_NBUF):
            off = last + b * _CHUNK
            pltpu.make_async_copy(table_hbm.at[idx_v[b]], rows_v[b], s_g[b]).wait()
            pltpu.async_copy(rows_v[b], out_hbm.at[pl.ds(off, _CHUNK)], s_o[b])
        for b in range(_NBUF):
            off = last + b * _CHUNK
            pltpu.make_async_copy(
                rows_v[b], out_hbm.at[pl.ds(off, _CHUNK)], s_o[b]
            ).wait()

    return gather_kernel


def kernel(indices, table):
    batch, seq = indices.shape
    vocab, dim = table.shape
    info = plsc.get_sparse_core_info()
    n_workers = info.num_cores * info.num_subcores
    flat = indices.reshape(-1)
    out = _make_gather(flat.shape[0], vocab, dim, n_workers, info.num_cores)(
        flat, table
    )
    return out.reshape(batch, seq, dim)
